# baseline (speedup 1.0000x reference)
import functools

import jax
import jax.numpy as jnp
from jax import lax
from jax.experimental import pallas as pl
from jax.experimental.pallas import tpu as pltpu
from jax.experimental.pallas import tpu_sc as plsc

N = 10000
E = 160000
G = 64
IN_DIM = 256
HID = 512
OUT = 256
NE = 6

CW = 128
NCH = HID // CW
NCO = OUT // CW

NC = 2
NS = 16
EPT = E // NS
KP = 128
NBP = 80
QB = 16
EPAD = NBP * KP - EPT
TR = 632
AN = 10008
TRL = N - (NS - 1) * TR
ZBR = 56

RB = 400
NRB = N // RB


def _mesh():
    return plsc.VectorSubcoreMesh(core_axis_name="c", subcore_axis_name="s")


@functools.cache
def _make_hist():
    nb2 = 50
    kk = 100

    scratch = [
        pltpu.VMEM((nb2, kk), jnp.int32),
        pltpu.VMEM((nb2, kk), jnp.int32),
        pltpu.VMEM((N,), jnp.float32),
        pltpu.VMEM((N,), jnp.float32),
    ]

    @functools.partial(
        pl.kernel,
        out_type=jax.ShapeDtypeStruct((2 * NC * NS, N), jnp.float32),
        mesh=_mesh(),
        scratch_types=scratch,
        compiler_params=pltpu.CompilerParams(needs_layout_passes=False),
    )
    def hist_kernel(src_ref, dst_ref, parts_ref, idx_a, idx_b, hs_loc, hd_loc):
        c = lax.axis_index("c")
        s = lax.axis_index("s")

        def zero_body(r, _):
            hs_loc[pl.ds(r * 16, 16)] = jnp.zeros((16,), jnp.float32)
            hd_loc[pl.ds(r * 16, 16)] = jnp.zeros((16,), jnp.float32)
            return 0

        lax.fori_loop(0, N // 16, zero_body, 0)

        w = c * NS + s
        pltpu.sync_copy(src_ref.at[w], idx_a)
        pltpu.sync_copy(dst_ref.at[w], idx_b)

        iot = lax.iota(jnp.int32, 16)
        tail_mask = iot >= 12
        ones16 = jnp.ones((16,), jnp.float32)

        def blk_body(b, _):
            for t in range(6):
                va = idx_a[b, pl.ds(t * 16, 16)]
                plsc.addupdate_scatter(hs_loc, [va], ones16)
                vb = idx_b[b, pl.ds(t * 16, 16)]
                plsc.addupdate_scatter(hd_loc, [vb], ones16)
            va = idx_a[b, pl.ds(kk - 16, 16)]
            plsc.addupdate_scatter(hs_loc, [va], ones16, mask=tail_mask)
            vb = idx_b[b, pl.ds(kk - 16, 16)]
            plsc.addupdate_scatter(hd_loc, [vb], ones16, mask=tail_mask)
            return 0

        lax.fori_loop(0, nb2, blk_body, 0)

        pltpu.sync_copy(hs_loc, parts_ref.at[w])
        pltpu.sync_copy(hd_loc, parts_ref.at[NC * NS + w])

    return hist_kernel


@functools.cache
def _make_segsum(n_ch, n_exp):
    cps = n_ch // NC

    scratch = [
        pltpu.VMEM((QB, KP), jnp.int32),
        pltpu.VMEM((QB, KP), jnp.int32),
        pltpu.VMEM((2 * KP, CW), jnp.float32),
        pltpu.VMEM((ZBR, CW), jnp.float32),
        pltpu.VMEM_SHARED((AN, CW), jnp.float32),
        pltpu.SemaphoreType.DMA,
        pltpu.SemaphoreType.DMA,
    ]
    out_type = tuple(
        jax.ShapeDtypeStruct((n_exp * N, CW), jnp.float32) for _ in range(n_ch)
    )

    @functools.partial(
        pl.kernel, out_type=out_type, mesh=_mesh(), scratch_types=scratch,
        compiler_params=pltpu.CompilerParams(needs_layout_passes=False),
    )
    def seg_kernel(*refs):
        h_refs = refs[:n_ch]
        src_ref = refs[n_ch]
        dst_ref = refs[n_ch + 1]
        out_refs = refs[n_ch + 2 : 2 * n_ch + 2]
        idx_s, idx_d, buf, zbuf, accum, semg, semsc = refs[2 * n_ch + 2 :]

        c = lax.axis_index("c")
        s = lax.axis_index("s")

        def zb(r, _):
            for j in range(CW // 16):
                zbuf[r, pl.ds(j * 16, 16)] = jnp.zeros((16,), jnp.float32)
            return 0

        lax.fori_loop(0, ZBR, zb, 0)

        for ci in range(n_ch):

            @pl.when(c == ci // cps)
            def _(ci=ci):
                h_hbm = h_refs[ci]
                out_hbm = out_refs[ci]

                def expert_body(e, _):
                    base = pl.multiple_of(s * TR, 8)

                    def zero_rows(total):
                        z0 = 0
                        while z0 < total:
                            zn = min(ZBR, total - z0)
                            pltpu.sync_copy(
                                zbuf.at[pl.ds(0, zn)],
                                accum.at[pl.ds(base + z0, zn)],
                            )
                            z0 += zn

                    @pl.when(s < NS - 1)
                    def _():
                        zero_rows(TR)

                    @pl.when(s == NS - 1)
                    def _():
                        zero_rows(AN - (NS - 1) * TR)

                    plsc.subcore_barrier()

                    def wait_pair(sem):
                        pltpu.make_async_copy(
                            h_hbm.at[pl.ds(0, 2 * KP)], buf, sem
                        ).wait()

                    lo = buf.at[pl.ds(0, KP)]
                    hi = buf.at[pl.ds(KP, KP)]

                    def blk(i, _):
                        b0 = 2 * i
                        b1 = b0 + 1

                        @pl.when(i > 0)
                        def _():
                            wait_pair(semsc)

                        pltpu.async_copy(h_hbm.at[idx_s.at[b0]], lo, semg)
                        pltpu.async_copy(h_hbm.at[idx_s.at[b1]], hi, semg)
                        wait_pair(semg)
                        pltpu.async_copy(
                            lo, accum.at[idx_d.at[b0]], semsc, add=True
                        )
                        pltpu.async_copy(
                            hi, accum.at[idx_d.at[b1]], semsc, add=True
                        )
                        return 0

                    for q in range(NBP // QB):
                        qoff = q * QB
                        pltpu.sync_copy(
                            src_ref.at[e * NS + s].at[pl.ds(qoff, QB)], idx_s
                        )
                        pltpu.sync_copy(
                            dst_ref.at[s].at[pl.ds(qoff, QB)], idx_d
                        )
                        lax.fori_loop(0, QB // 2, blk, 0)
                        wait_pair(semsc)
                    plsc.subcore_barrier()
                    obase = pl.multiple_of(e * N + base, 8)

                    @pl.when(s < NS - 1)
                    def _():
                        pltpu.sync_copy(
                            accum.at[pl.ds(base, TR)],
                            out_hbm.at[pl.ds(obase, TR)],
                        )

                    @pl.when(s == NS - 1)
                    def _():
                        pltpu.sync_copy(
                            accum.at[pl.ds(base, TRL)],
                            out_hbm.at[pl.ds(obase, TRL)],
                        )

                    return 0

                lax.fori_loop(0, n_exp, expert_body, 0)

    return seg_kernel


def _segsum(h_chunks, src_idx, dst_idx, n_exp):
    fn = _make_segsum(len(h_chunks), n_exp)
    return fn(*h_chunks, src_idx, dst_idx)


def _enc_body(x_ref, w_ref, b_ref, *outs):
    h = jnp.dot(x_ref[...], w_ref[...], preferred_element_type=jnp.float32)
    h = jnp.maximum(h + b_ref[...], 0.0)
    for ci in range(NCH):
        outs[ci][...] = h[:, ci * CW : (ci + 1) * CW]


def _encoder(x, w, b):
    return pl.pallas_call(
        _enc_body,
        grid=(NRB,),
        in_specs=[
            pl.BlockSpec((RB, IN_DIM), lambda i: (i, 0)),
            pl.BlockSpec((IN_DIM, HID), lambda i: (0, 0)),
            pl.BlockSpec((1, HID), lambda i: (0, 0)),
        ],
        out_specs=[pl.BlockSpec((RB, CW), lambda i: (i, 0))] * NCH,
        out_shape=[jax.ShapeDtypeStruct((N, CW), jnp.float32)] * NCH,
    )(x, w, b)


def _counts_body(parts_ref, batch_ref, counts_ref, deg_ref):
    p = parts_ref[...].reshape(2 * NC * NS, RB)
    half = NC * NS
    sel_out = (lax.broadcasted_iota(jnp.int32, (2 * half, 1), 0) < half).astype(
        jnp.float32
    )
    sel_in = 1.0 - sel_out
    dims = (((0,), (0,)), ((), ()))
    outd = lax.dot_general(p, sel_out, dims, preferred_element_type=jnp.float32)
    ind = lax.dot_general(p, sel_in, dims, preferred_element_type=jnp.float32)
    deg_ref[...] = jnp.maximum(ind, 1.0)
    bt = batch_ref[0]
    oh = (bt == lax.broadcasted_iota(jnp.int32, (RB, G), 1)).astype(jnp.float32)
    nodes_c = jnp.sum(oh, axis=0, keepdims=True)
    edges_c = lax.dot_general(outd, oh, dims, preferred_element_type=jnp.float32)
    contrib = jnp.concatenate(
        [nodes_c, edges_c, jnp.zeros((6, G), jnp.float32)], axis=0
    )

    @pl.when(pl.program_id(0) == 0)
    def _():
        counts_ref[...] = jnp.zeros((8, G), jnp.float32)

    counts_ref[...] += contrib


def _counts(parts, batch_c):
    return pl.pallas_call(
        _counts_body,
        grid=(NRB,),
        in_specs=[
            pl.BlockSpec((2 * NC * NS, 1, 1, RB), lambda i: (0, i, 0, 0)),
            pl.BlockSpec((1, RB, 1), lambda i: (i, 0, 0)),
        ],
        out_specs=[
            pl.BlockSpec((8, G), lambda i: (0, 0)),
            pl.BlockSpec((RB, 1), lambda i: (i, 0)),
        ],
        out_shape=[
            jax.ShapeDtypeStruct((8, G), jnp.float32),
            jax.ShapeDtypeStruct((N, 1), jnp.float32),
        ],
    )(parts.reshape(2 * NC * NS, NRB, 1, RB), batch_c)


def _router_body(*refs):
    hrefs = refs[:NCH]
    (batch_ref, counts_ref, w1h_ref, w1s_ref, b1_ref, w2_ref, b2_ref,
     gates_ref) = refs[NCH:]
    cnt = counts_ref[...]
    lp = jnp.log1p(cnt[0:2])
    bt = batch_ref[0]
    oh = (bt == lax.broadcasted_iota(jnp.int32, (RB, G), 1)).astype(jnp.float32)
    sf = lax.dot_general(
        oh, lp, (((1,), (1,)), ((), ())), preferred_element_type=jnp.float32
    )
    w1h = w1h_ref[...]
    acc = b1_ref[...] + jnp.dot(
        sf, w1s_ref[...], preferred_element_type=jnp.float32
    )
    for ci in range(NCH):
        acc = acc + jnp.dot(
            hrefs[ci][...],
            w1h[ci * CW : (ci + 1) * CW],
            preferred_element_type=jnp.float32,
        )
    r1 = jnp.maximum(acc, 0.0)
    logits = (
        jnp.dot(r1, w2_ref[...], preferred_element_type=jnp.float32)
        + b2_ref[...]
    )
    iot = lax.broadcasted_iota(jnp.int32, (RB, NE), 1)
    m1 = jnp.max(logits, axis=1, keepdims=True)
    i1 = jnp.min(jnp.where(logits == m1, iot, NE), axis=1, keepdims=True)
    l2 = jnp.where(iot == i1, -jnp.inf, logits)
    m2 = jnp.max(l2, axis=1, keepdims=True)
    i2 = jnp.min(jnp.where(l2 == m2, iot, NE), axis=1, keepdims=True)
    e2 = jnp.exp(m2 - m1)
    den = 1.0 + e2
    g1 = 1.0 / den
    g2 = e2 / den
    gates_ref[...] = jnp.where(iot == i1, g1, 0.0) + jnp.where(
        iot == i2, g2, 0.0
    )


def _router(hc, batch_c, counts, w1h, w1s, b1, w2, b2):
    return pl.pallas_call(
        _router_body,
        grid=(NRB,),
        in_specs=[pl.BlockSpec((RB, CW), lambda i: (i, 0))] * NCH
        + [
            pl.BlockSpec((1, RB, 1), lambda i: (i, 0, 0)),
            pl.BlockSpec((8, G), lambda i: (0, 0)),
            pl.BlockSpec((HID, HID), lambda i: (0, 0)),
            pl.BlockSpec((2, HID), lambda i: (0, 0)),
            pl.BlockSpec((1, HID), lambda i: (0, 0)),
            pl.BlockSpec((HID, NE), lambda i: (0, 0)),
            pl.BlockSpec((1, NE), lambda i: (0, 0)),
        ],
        out_specs=pl.BlockSpec((RB, NE), lambda i: (i, 0)),
        out_shape=jax.ShapeDtypeStruct((N, NE), jnp.float32),
    )(*hc, batch_c, counts, w1h, w1s, b1, w2, b2)


EH = 3


@functools.cache
def _make_layer(multi_h, relu, n_e, e0):
    def body(*refs):
        hrefs = refs[:NCH]
        arefs = refs[NCH : 2 * NCH]
        deg_ref, ws_ref, wn_ref, b_ref = refs[2 * NCH : 2 * NCH + 4]
        outs = refs[2 * NCH + 4 :]
        dinv = 1.0 / deg_ref[...]
        ws = ws_ref[0]
        wn = wn_ref[0]
        acc = b_ref[0]
        for ci in range(NCH):
            hcv = hrefs[ci][0]
            acv = arefs[ci][0] * dinv
            acc = acc + jnp.dot(
                hcv, ws[ci * CW : (ci + 1) * CW],
                preferred_element_type=jnp.float32,
            )
            acc = acc + jnp.dot(
                acv, wn[ci * CW : (ci + 1) * CW],
                preferred_element_type=jnp.float32,
            )
        if relu:
            acc = jnp.maximum(acc, 0.0)
        for ci in range(NCH):
            outs[ci][...] = acc[:, ci * CW : (ci + 1) * CW].reshape(1, RB, CW)

    def h_map(e, i):
        return (e if multi_h else 0, i, 0)

    return pl.pallas_call(
        body,
        grid=(n_e, NRB),
        in_specs=[pl.BlockSpec((1, RB, CW), h_map)] * (2 * NCH)
        + [
            pl.BlockSpec((RB, 1), lambda e, i: (i, 0)),
            pl.BlockSpec((1, HID, HID), lambda e, i: (e + e0, 0, 0)),
            pl.BlockSpec((1, HID, HID), lambda e, i: (e + e0, 0, 0)),
            pl.BlockSpec((1, 1, HID), lambda e, i: (e + e0, 0, 0)),
        ],
        out_specs=[pl.BlockSpec((1, RB, CW), lambda e, i: (e, i, 0))] * NCH,
        out_shape=[jax.ShapeDtypeStruct((n_e, N, CW), jnp.float32)] * NCH,
    )


def _layer(hc, ac, deg, ws, wn, b, multi_h, n_e=NE, e0=0, relu=True):
    fn = _make_layer(multi_h, relu, n_e, e0)
    return fn(*hc, *ac, deg, ws, wn, b)


def _layer3_body(*refs):
    hrefs = refs[:NCH]
    ws_ref, wn_ref, b_ref, hs_ref = refs[NCH : NCH + 4]
    nouts = refs[NCH + 4 :]
    ws = ws_ref[0]
    wn = wn_ref[0]
    hs = b_ref[0]
    hn = jnp.zeros((RB, OUT), jnp.float32)
    for ci in range(NCH):
        hcv = hrefs[ci][0]
        hs = hs + jnp.dot(
            hcv, ws[ci * CW : (ci + 1) * CW],
            preferred_element_type=jnp.float32,
        )
        hn = hn + jnp.dot(
            hcv, wn[ci * CW : (ci + 1) * CW],
            preferred_element_type=jnp.float32,
        )
    hs_ref[...] = hs.reshape(1, RB, OUT)
    for ci in range(NCO):
        nouts[ci][...] = hn[:, ci * CW : (ci + 1) * CW].reshape(1, RB, CW)


@functools.cache
def _make_layer3(n_e, e0):
    return pl.pallas_call(
        _layer3_body,
        grid=(n_e, NRB),
        in_specs=[pl.BlockSpec((1, RB, CW), lambda e, i: (e, i, 0))] * NCH
        + [
            pl.BlockSpec((1, HID, OUT), lambda e, i: (e + e0, 0, 0)),
            pl.BlockSpec((1, HID, OUT), lambda e, i: (e + e0, 0, 0)),
            pl.BlockSpec((1, 1, OUT), lambda e, i: (e + e0, 0, 0)),
        ],
        out_specs=[pl.BlockSpec((1, RB, OUT), lambda e, i: (e, i, 0))]
        + [pl.BlockSpec((1, RB, CW), lambda e, i: (e, i, 0))] * NCO,
        out_shape=[jax.ShapeDtypeStruct((n_e, N, OUT), jnp.float32)]
        + [jax.ShapeDtypeStruct((n_e, N, CW), jnp.float32)] * NCO,
    )


def _layer3(hc, ws, wn, b, n_e=NE, e0=0):
    return _make_layer3(n_e, e0)(*hc, ws, wn, b)


def _final_body(*refs):
    gates_ref, deg_ref = refs[:2]
    halves = []
    pos = 2
    for _ in range(NE // EH):
        halves.append((refs[pos], refs[pos + 1 : pos + 1 + NCO]))
        pos += 1 + NCO
    y_ref = refs[pos]
    dinv = 1.0 / deg_ref[...]
    g = gates_ref[...]
    acc = jnp.zeros((RB, OUT), jnp.float32)
    for hi, (hs_ref, arefs) in enumerate(halves):
        for el in range(EH):
            e = hi * EH + el
            aggc = jnp.concatenate([a[el] for a in arefs], axis=1) * dinv
            acc = acc + g[:, e : e + 1] * (hs_ref[el] + aggc)
    y_ref[...] = acc


def _final(gates, deg, parts3):
    args = [gates, deg]
    in_specs = [
        pl.BlockSpec((RB, NE), lambda i: (i, 0)),
        pl.BlockSpec((RB, 1), lambda i: (i, 0)),
    ]
    for hs3, aggs in parts3:
        args.append(hs3)
        in_specs.append(pl.BlockSpec((EH, RB, OUT), lambda i: (0, i, 0)))
        args.extend(aggs)
        in_specs.extend([pl.BlockSpec((EH, RB, CW), lambda i: (0, i, 0))] * NCO)
    return pl.pallas_call(
        _final_body,
        grid=(NRB,),
        in_specs=in_specs,
        out_specs=pl.BlockSpec((RB, OUT), lambda i: (i, 0)),
        out_shape=jax.ShapeDtypeStruct((N, OUT), jnp.float32),
    )(*args)


def kernel(x, edge_index, batch, W_enc, b_enc, W_r1, b_r1, W_r2, b_r2,
           Wself1, Wneigh1, bl1, Wself2, Wneigh2, bl2, Wself3, Wneigh3, bl3):
    src = edge_index[0]
    dst = edge_index[1]
    srcm = src.reshape(NS, EPT)
    dstm = dst.reshape(NS, EPT)
    src_slab = jnp.concatenate(
        [srcm, jnp.zeros((NS, EPAD), jnp.int32)], axis=1
    ).reshape(NS, NBP, KP)
    dst_slab = jnp.concatenate(
        [dstm, jnp.full((NS, EPAD), N, jnp.int32)], axis=1
    ).reshape(NS, NBP, KP)
    off = (jnp.arange(EH, dtype=jnp.int32) * N)[:, None, None]
    src3 = jnp.concatenate(
        [srcm[None] + off, jnp.zeros((EH, NS, EPAD), jnp.int32)], axis=2
    ).reshape(EH * NS, NBP, KP)
    batch_c = batch.reshape(NRB, RB, 1)

    parts = _make_hist()(src.reshape(NC * NS, 50, 100),
                         dst.reshape(NC * NS, 50, 100))
    hc = _encoder(x, W_enc, b_enc.reshape(1, HID))
    counts, deg = _counts(parts, batch_c)
    gates = _router(hc, batch_c, counts, W_r1[:HID], W_r1[HID:],
                    b_r1.reshape(1, HID), W_r2, b_r2.reshape(1, NE))

    a1 = _segsum(hc, src_slab, dst_slab, 1)
    hc1 = [h.reshape(1, N, CW) for h in hc]
    a1r = [a.reshape(1, N, CW) for a in a1]
    bl1r = bl1.reshape(NE, 1, HID)
    bl2r = bl2.reshape(NE, 1, HID)
    bl3r = bl3.reshape(NE, 1, OUT)

    parts3 = []
    for e0 in (0, EH):
        he1 = _layer(hc1, a1r, deg, Wself1, Wneigh1, bl1r,
                     multi_h=False, n_e=EH, e0=e0)
        a2 = _segsum([h.reshape(EH * N, CW) for h in he1], src3, dst_slab, EH)
        he2 = _layer(he1, [a.reshape(EH, N, CW) for a in a2], deg,
                     Wself2, Wneigh2, bl2r, multi_h=True, n_e=EH, e0=e0)
        l3 = _layer3(he2, Wself3, Wneigh3, bl3r, n_e=EH, e0=e0)
        a3 = _segsum([h.reshape(EH * N, CW) for h in l3[1:]],
                     src3, dst_slab, EH)
        parts3.append((l3[0], [a.reshape(EH, N, CW) for a in a3]))

    return _final(gates, deg, parts3)


# speedup vs baseline: 1.0097x; 1.0097x over previous
import functools

import jax
import jax.numpy as jnp
from jax import lax
from jax.experimental import pallas as pl
from jax.experimental.pallas import tpu as pltpu
from jax.experimental.pallas import tpu_sc as plsc

N = 10000
E = 160000
G = 64
IN_DIM = 256
HID = 512
OUT = 256
NE = 6

CW = 128
NCH = HID // CW
NCO = OUT // CW

NC = 2
NS = 16
EPT = E // NS
KP = 128
NBP = 80
QB = 16
EPAD = NBP * KP - EPT
TR = 632
AN = 10008
TRL = N - (NS - 1) * TR
ZBR = 56

RB = 400
NRB = N // RB


def _mesh():
    return plsc.VectorSubcoreMesh(core_axis_name="c", subcore_axis_name="s")


@functools.cache
def _make_hist():
    nb2 = 50
    kk = 100

    scratch = [
        pltpu.VMEM((nb2, kk), jnp.int32),
        pltpu.VMEM((nb2, kk), jnp.int32),
        pltpu.VMEM((N,), jnp.float32),
        pltpu.VMEM((N,), jnp.float32),
    ]

    @functools.partial(
        pl.kernel,
        out_type=jax.ShapeDtypeStruct((2 * NC * NS, N), jnp.float32),
        mesh=_mesh(),
        scratch_types=scratch,
        compiler_params=pltpu.CompilerParams(needs_layout_passes=False),
    )
    def hist_kernel(src_ref, dst_ref, parts_ref, idx_a, idx_b, hs_loc, hd_loc):
        c = lax.axis_index("c")
        s = lax.axis_index("s")

        def zero_body(r, _):
            hs_loc[pl.ds(r * 16, 16)] = jnp.zeros((16,), jnp.float32)
            hd_loc[pl.ds(r * 16, 16)] = jnp.zeros((16,), jnp.float32)
            return 0

        lax.fori_loop(0, N // 16, zero_body, 0)

        w = c * NS + s
        pltpu.sync_copy(src_ref.at[w], idx_a)
        pltpu.sync_copy(dst_ref.at[w], idx_b)

        iot = lax.iota(jnp.int32, 16)
        tail_mask = iot >= 12
        ones16 = jnp.ones((16,), jnp.float32)

        def blk_body(b, _):
            for t in range(6):
                va = idx_a[b, pl.ds(t * 16, 16)]
                plsc.addupdate_scatter(hs_loc, [va], ones16)
                vb = idx_b[b, pl.ds(t * 16, 16)]
                plsc.addupdate_scatter(hd_loc, [vb], ones16)
            va = idx_a[b, pl.ds(kk - 16, 16)]
            plsc.addupdate_scatter(hs_loc, [va], ones16, mask=tail_mask)
            vb = idx_b[b, pl.ds(kk - 16, 16)]
            plsc.addupdate_scatter(hd_loc, [vb], ones16, mask=tail_mask)
            return 0

        lax.fori_loop(0, nb2, blk_body, 0)

        pltpu.sync_copy(hs_loc, parts_ref.at[w])
        pltpu.sync_copy(hd_loc, parts_ref.at[NC * NS + w])

    return hist_kernel


@functools.cache
def _make_segsum(n_ch, n_exp):
    cps = n_ch // NC

    scratch = [
        pltpu.VMEM((QB, KP), jnp.int32),
        pltpu.VMEM((QB, KP), jnp.int32),
        pltpu.VMEM((KP, CW), jnp.float32),
        pltpu.VMEM((KP, CW), jnp.float32),
        pltpu.VMEM((ZBR, CW), jnp.float32),
        pltpu.VMEM_SHARED((AN, CW), jnp.float32),
        pltpu.SemaphoreType.DMA,
        pltpu.SemaphoreType.DMA,
        pltpu.SemaphoreType.DMA,
        pltpu.SemaphoreType.DMA,
    ]
    out_type = tuple(
        jax.ShapeDtypeStruct((n_exp * N, CW), jnp.float32) for _ in range(n_ch)
    )

    @functools.partial(
        pl.kernel, out_type=out_type, mesh=_mesh(), scratch_types=scratch,
        compiler_params=pltpu.CompilerParams(needs_layout_passes=False),
    )
    def seg_kernel(*refs):
        h_refs = refs[:n_ch]
        src_ref = refs[n_ch]
        dst_ref = refs[n_ch + 1]
        out_refs = refs[n_ch + 2 : 2 * n_ch + 2]
        (idx_s, idx_d, buf0, buf1, zbuf, accum,
         semg0, semg1, semsc0, semsc1) = refs[2 * n_ch + 2 :]

        c = lax.axis_index("c")
        s = lax.axis_index("s")

        def zb(r, _):
            for j in range(CW // 16):
                zbuf[r, pl.ds(j * 16, 16)] = jnp.zeros((16,), jnp.float32)
            return 0

        lax.fori_loop(0, ZBR, zb, 0)

        for ci in range(n_ch):

            @pl.when(c == ci // cps)
            def _(ci=ci):
                h_hbm = h_refs[ci]
                out_hbm = out_refs[ci]

                def expert_body(e, _):
                    base = pl.multiple_of(s * TR, 8)

                    def zero_rows(total):
                        z0 = 0
                        while z0 < total:
                            zn = min(ZBR, total - z0)
                            pltpu.sync_copy(
                                zbuf.at[pl.ds(0, zn)],
                                accum.at[pl.ds(base + z0, zn)],
                            )
                            z0 += zn

                    @pl.when(s < NS - 1)
                    def _():
                        zero_rows(TR)

                    @pl.when(s == NS - 1)
                    def _():
                        zero_rows(AN - (NS - 1) * TR)

                    plsc.subcore_barrier()

                    def wait_on(sem, bufref):
                        pltpu.make_async_copy(
                            h_hbm.at[pl.ds(0, KP)], bufref, sem
                        ).wait()

                    def blk(i, _):
                        b0 = 2 * i
                        b1 = b0 + 1

                        @pl.when(i > 0)
                        def _():
                            wait_on(semsc1, buf1)

                        pltpu.async_copy(h_hbm.at[idx_s.at[b1]], buf1, semg1)
                        wait_on(semg0, buf0)
                        pltpu.async_copy(
                            buf0, accum.at[idx_d.at[b0]], semsc0, add=True
                        )
                        wait_on(semg1, buf1)
                        pltpu.async_copy(
                            buf1, accum.at[idx_d.at[b1]], semsc1, add=True
                        )

                        @pl.when(i < QB // 2 - 1)
                        def _():
                            wait_on(semsc0, buf0)
                            pltpu.async_copy(
                                h_hbm.at[idx_s.at[b0 + 2]], buf0, semg0
                            )

                        return 0

                    for q in range(NBP // QB):
                        qoff = q * QB
                        pltpu.sync_copy(
                            src_ref.at[e * NS + s].at[pl.ds(qoff, QB)], idx_s
                        )
                        pltpu.sync_copy(
                            dst_ref.at[s].at[pl.ds(qoff, QB)], idx_d
                        )
                        pltpu.async_copy(h_hbm.at[idx_s.at[0]], buf0, semg0)
                        lax.fori_loop(0, QB // 2, blk, 0)
                        wait_on(semsc0, buf0)
                        wait_on(semsc1, buf1)
                    plsc.subcore_barrier()
                    obase = pl.multiple_of(e * N + base, 8)

                    @pl.when(s < NS - 1)
                    def _():
                        pltpu.sync_copy(
                            accum.at[pl.ds(base, TR)],
                            out_hbm.at[pl.ds(obase, TR)],
                        )

                    @pl.when(s == NS - 1)
                    def _():
                        pltpu.sync_copy(
                            accum.at[pl.ds(base, TRL)],
                            out_hbm.at[pl.ds(obase, TRL)],
                        )

                    return 0

                lax.fori_loop(0, n_exp, expert_body, 0)

    return seg_kernel


def _segsum(h_chunks, src_idx, dst_idx, n_exp):
    fn = _make_segsum(len(h_chunks), n_exp)
    return fn(*h_chunks, src_idx, dst_idx)


def _enc_body(x_ref, w_ref, b_ref, *outs):
    h = jnp.dot(x_ref[...], w_ref[...], preferred_element_type=jnp.float32)
    h = jnp.maximum(h + b_ref[...], 0.0)
    for ci in range(NCH):
        outs[ci][...] = h[:, ci * CW : (ci + 1) * CW]


def _encoder(x, w, b):
    return pl.pallas_call(
        _enc_body,
        grid=(NRB,),
        in_specs=[
            pl.BlockSpec((RB, IN_DIM), lambda i: (i, 0)),
            pl.BlockSpec((IN_DIM, HID), lambda i: (0, 0)),
            pl.BlockSpec((1, HID), lambda i: (0, 0)),
        ],
        out_specs=[pl.BlockSpec((RB, CW), lambda i: (i, 0))] * NCH,
        out_shape=[jax.ShapeDtypeStruct((N, CW), jnp.float32)] * NCH,
    )(x, w, b)


def _counts_body(parts_ref, batch_ref, counts_ref, deg_ref):
    p = parts_ref[...].reshape(2 * NC * NS, RB)
    half = NC * NS
    sel_out = (lax.broadcasted_iota(jnp.int32, (2 * half, 1), 0) < half).astype(
        jnp.float32
    )
    sel_in = 1.0 - sel_out
    dims = (((0,), (0,)), ((), ()))
    outd = lax.dot_general(p, sel_out, dims, preferred_element_type=jnp.float32)
    ind = lax.dot_general(p, sel_in, dims, preferred_element_type=jnp.float32)
    deg_ref[...] = jnp.maximum(ind, 1.0)
    bt = batch_ref[0]
    oh = (bt == lax.broadcasted_iota(jnp.int32, (RB, G), 1)).astype(jnp.float32)
    nodes_c = jnp.sum(oh, axis=0, keepdims=True)
    edges_c = lax.dot_general(outd, oh, dims, preferred_element_type=jnp.float32)
    contrib = jnp.concatenate(
        [nodes_c, edges_c, jnp.zeros((6, G), jnp.float32)], axis=0
    )

    @pl.when(pl.program_id(0) == 0)
    def _():
        counts_ref[...] = jnp.zeros((8, G), jnp.float32)

    counts_ref[...] += contrib


def _counts(parts, batch_c):
    return pl.pallas_call(
        _counts_body,
        grid=(NRB,),
        in_specs=[
            pl.BlockSpec((2 * NC * NS, 1, 1, RB), lambda i: (0, i, 0, 0)),
            pl.BlockSpec((1, RB, 1), lambda i: (i, 0, 0)),
        ],
        out_specs=[
            pl.BlockSpec((8, G), lambda i: (0, 0)),
            pl.BlockSpec((RB, 1), lambda i: (i, 0)),
        ],
        out_shape=[
            jax.ShapeDtypeStruct((8, G), jnp.float32),
            jax.ShapeDtypeStruct((N, 1), jnp.float32),
        ],
    )(parts.reshape(2 * NC * NS, NRB, 1, RB), batch_c)


def _router_body(*refs):
    hrefs = refs[:NCH]
    (batch_ref, counts_ref, w1h_ref, w1s_ref, b1_ref, w2_ref, b2_ref,
     gates_ref) = refs[NCH:]
    cnt = counts_ref[...]
    lp = jnp.log1p(cnt[0:2])
    bt = batch_ref[0]
    oh = (bt == lax.broadcasted_iota(jnp.int32, (RB, G), 1)).astype(jnp.float32)
    sf = lax.dot_general(
        oh, lp, (((1,), (1,)), ((), ())), preferred_element_type=jnp.float32
    )
    w1h = w1h_ref[...]
    acc = b1_ref[...] + jnp.dot(
        sf, w1s_ref[...], preferred_element_type=jnp.float32
    )
    for ci in range(NCH):
        acc = acc + jnp.dot(
            hrefs[ci][...],
            w1h[ci * CW : (ci + 1) * CW],
            preferred_element_type=jnp.float32,
        )
    r1 = jnp.maximum(acc, 0.0)
    logits = (
        jnp.dot(r1, w2_ref[...], preferred_element_type=jnp.float32)
        + b2_ref[...]
    )
    iot = lax.broadcasted_iota(jnp.int32, (RB, NE), 1)
    m1 = jnp.max(logits, axis=1, keepdims=True)
    i1 = jnp.min(jnp.where(logits == m1, iot, NE), axis=1, keepdims=True)
    l2 = jnp.where(iot == i1, -jnp.inf, logits)
    m2 = jnp.max(l2, axis=1, keepdims=True)
    i2 = jnp.min(jnp.where(l2 == m2, iot, NE), axis=1, keepdims=True)
    e2 = jnp.exp(m2 - m1)
    den = 1.0 + e2
    g1 = 1.0 / den
    g2 = e2 / den
    gates_ref[...] = jnp.where(iot == i1, g1, 0.0) + jnp.where(
        iot == i2, g2, 0.0
    )


def _router(hc, batch_c, counts, w1h, w1s, b1, w2, b2):
    return pl.pallas_call(
        _router_body,
        grid=(NRB,),
        in_specs=[pl.BlockSpec((RB, CW), lambda i: (i, 0))] * NCH
        + [
            pl.BlockSpec((1, RB, 1), lambda i: (i, 0, 0)),
            pl.BlockSpec((8, G), lambda i: (0, 0)),
            pl.BlockSpec((HID, HID), lambda i: (0, 0)),
            pl.BlockSpec((2, HID), lambda i: (0, 0)),
            pl.BlockSpec((1, HID), lambda i: (0, 0)),
            pl.BlockSpec((HID, NE), lambda i: (0, 0)),
            pl.BlockSpec((1, NE), lambda i: (0, 0)),
        ],
        out_specs=pl.BlockSpec((RB, NE), lambda i: (i, 0)),
        out_shape=jax.ShapeDtypeStruct((N, NE), jnp.float32),
    )(*hc, batch_c, counts, w1h, w1s, b1, w2, b2)


EH = 3


@functools.cache
def _make_layer(multi_h, relu, n_e, e0):
    def body(*refs):
        hrefs = refs[:NCH]
        arefs = refs[NCH : 2 * NCH]
        deg_ref, ws_ref, wn_ref, b_ref = refs[2 * NCH : 2 * NCH + 4]
        outs = refs[2 * NCH + 4 :]
        dinv = 1.0 / deg_ref[...]
        ws = ws_ref[0]
        wn = wn_ref[0]
        acc = b_ref[0]
        for ci in range(NCH):
            hcv = hrefs[ci][0]
            acv = arefs[ci][0] * dinv
            acc = acc + jnp.dot(
                hcv, ws[ci * CW : (ci + 1) * CW],
                preferred_element_type=jnp.float32,
            )
            acc = acc + jnp.dot(
                acv, wn[ci * CW : (ci + 1) * CW],
                preferred_element_type=jnp.float32,
            )
        if relu:
            acc = jnp.maximum(acc, 0.0)
        for ci in range(NCH):
            outs[ci][...] = acc[:, ci * CW : (ci + 1) * CW].reshape(1, RB, CW)

    def h_map(e, i):
        return (e if multi_h else 0, i, 0)

    return pl.pallas_call(
        body,
        grid=(n_e, NRB),
        in_specs=[pl.BlockSpec((1, RB, CW), h_map)] * (2 * NCH)
        + [
            pl.BlockSpec((RB, 1), lambda e, i: (i, 0)),
            pl.BlockSpec((1, HID, HID), lambda e, i: (e + e0, 0, 0)),
            pl.BlockSpec((1, HID, HID), lambda e, i: (e + e0, 0, 0)),
            pl.BlockSpec((1, 1, HID), lambda e, i: (e + e0, 0, 0)),
        ],
        out_specs=[pl.BlockSpec((1, RB, CW), lambda e, i: (e, i, 0))] * NCH,
        out_shape=[jax.ShapeDtypeStruct((n_e, N, CW), jnp.float32)] * NCH,
    )


def _layer(hc, ac, deg, ws, wn, b, multi_h, n_e=NE, e0=0, relu=True):
    fn = _make_layer(multi_h, relu, n_e, e0)
    return fn(*hc, *ac, deg, ws, wn, b)


def _layer3_body(*refs):
    hrefs = refs[:NCH]
    ws_ref, wn_ref, b_ref, hs_ref = refs[NCH : NCH + 4]
    nouts = refs[NCH + 4 :]
    ws = ws_ref[0]
    wn = wn_ref[0]
    hs = b_ref[0]
    hn = jnp.zeros((RB, OUT), jnp.float32)
    for ci in range(NCH):
        hcv = hrefs[ci][0]
        hs = hs + jnp.dot(
            hcv, ws[ci * CW : (ci + 1) * CW],
            preferred_element_type=jnp.float32,
        )
        hn = hn + jnp.dot(
            hcv, wn[ci * CW : (ci + 1) * CW],
            preferred_element_type=jnp.float32,
        )
    hs_ref[...] = hs.reshape(1, RB, OUT)
    for ci in range(NCO):
        nouts[ci][...] = hn[:, ci * CW : (ci + 1) * CW].reshape(1, RB, CW)


@functools.cache
def _make_layer3(n_e, e0):
    return pl.pallas_call(
        _layer3_body,
        grid=(n_e, NRB),
        in_specs=[pl.BlockSpec((1, RB, CW), lambda e, i: (e, i, 0))] * NCH
        + [
            pl.BlockSpec((1, HID, OUT), lambda e, i: (e + e0, 0, 0)),
            pl.BlockSpec((1, HID, OUT), lambda e, i: (e + e0, 0, 0)),
            pl.BlockSpec((1, 1, OUT), lambda e, i: (e + e0, 0, 0)),
        ],
        out_specs=[pl.BlockSpec((1, RB, OUT), lambda e, i: (e, i, 0))]
        + [pl.BlockSpec((1, RB, CW), lambda e, i: (e, i, 0))] * NCO,
        out_shape=[jax.ShapeDtypeStruct((n_e, N, OUT), jnp.float32)]
        + [jax.ShapeDtypeStruct((n_e, N, CW), jnp.float32)] * NCO,
    )


def _layer3(hc, ws, wn, b, n_e=NE, e0=0):
    return _make_layer3(n_e, e0)(*hc, ws, wn, b)


def _final_body(*refs):
    gates_ref, deg_ref = refs[:2]
    halves = []
    pos = 2
    for _ in range(NE // EH):
        halves.append((refs[pos], refs[pos + 1 : pos + 1 + NCO]))
        pos += 1 + NCO
    y_ref = refs[pos]
    dinv = 1.0 / deg_ref[...]
    g = gates_ref[...]
    acc = jnp.zeros((RB, OUT), jnp.float32)
    for hi, (hs_ref, arefs) in enumerate(halves):
        for el in range(EH):
            e = hi * EH + el
            aggc = jnp.concatenate([a[el] for a in arefs], axis=1) * dinv
            acc = acc + g[:, e : e + 1] * (hs_ref[el] + aggc)
    y_ref[...] = acc


def _final(gates, deg, parts3):
    args = [gates, deg]
    in_specs = [
        pl.BlockSpec((RB, NE), lambda i: (i, 0)),
        pl.BlockSpec((RB, 1), lambda i: (i, 0)),
    ]
    for hs3, aggs in parts3:
        args.append(hs3)
        in_specs.append(pl.BlockSpec((EH, RB, OUT), lambda i: (0, i, 0)))
        args.extend(aggs)
        in_specs.extend([pl.BlockSpec((EH, RB, CW), lambda i: (0, i, 0))] * NCO)
    return pl.pallas_call(
        _final_body,
        grid=(NRB,),
        in_specs=in_specs,
        out_specs=pl.BlockSpec((RB, OUT), lambda i: (i, 0)),
        out_shape=jax.ShapeDtypeStruct((N, OUT), jnp.float32),
    )(*args)


def kernel(x, edge_index, batch, W_enc, b_enc, W_r1, b_r1, W_r2, b_r2,
           Wself1, Wneigh1, bl1, Wself2, Wneigh2, bl2, Wself3, Wneigh3, bl3):
    src = edge_index[0]
    dst = edge_index[1]
    srcm = src.reshape(NS, EPT)
    dstm = dst.reshape(NS, EPT)
    src_slab = jnp.concatenate(
        [srcm, jnp.zeros((NS, EPAD), jnp.int32)], axis=1
    ).reshape(NS, NBP, KP)
    dst_slab = jnp.concatenate(
        [dstm, jnp.full((NS, EPAD), N, jnp.int32)], axis=1
    ).reshape(NS, NBP, KP)
    off = (jnp.arange(EH, dtype=jnp.int32) * N)[:, None, None]
    src3 = jnp.concatenate(
        [srcm[None] + off, jnp.zeros((EH, NS, EPAD), jnp.int32)], axis=2
    ).reshape(EH * NS, NBP, KP)
    batch_c = batch.reshape(NRB, RB, 1)

    parts = _make_hist()(src.reshape(NC * NS, 50, 100),
                         dst.reshape(NC * NS, 50, 100))
    hc = _encoder(x, W_enc, b_enc.reshape(1, HID))
    counts, deg = _counts(parts, batch_c)
    gates = _router(hc, batch_c, counts, W_r1[:HID], W_r1[HID:],
                    b_r1.reshape(1, HID), W_r2, b_r2.reshape(1, NE))

    a1 = _segsum(hc, src_slab, dst_slab, 1)
    hc1 = [h.reshape(1, N, CW) for h in hc]
    a1r = [a.reshape(1, N, CW) for a in a1]
    bl1r = bl1.reshape(NE, 1, HID)
    bl2r = bl2.reshape(NE, 1, HID)
    bl3r = bl3.reshape(NE, 1, OUT)

    parts3 = []
    for e0 in (0, EH):
        he1 = _layer(hc1, a1r, deg, Wself1, Wneigh1, bl1r,
                     multi_h=False, n_e=EH, e0=e0)
        a2 = _segsum([h.reshape(EH * N, CW) for h in he1], src3, dst_slab, EH)
        he2 = _layer(he1, [a.reshape(EH, N, CW) for a in a2], deg,
                     Wself2, Wneigh2, bl2r, multi_h=True, n_e=EH, e0=e0)
        l3 = _layer3(he2, Wself3, Wneigh3, bl3r, n_e=EH, e0=e0)
        a3 = _segsum([h.reshape(EH * N, CW) for h in l3[1:]],
                     src3, dst_slab, EH)
        parts3.append((l3[0], [a.reshape(EH, N, CW) for a in a3]))

    return _final(gates, deg, parts3)
